# baseline (speedup 1.0000x reference)
import functools

import jax
import jax.numpy as jnp
from jax import lax
from jax.experimental import pallas as pl
from jax.experimental.pallas import tpu as pltpu
from jax.experimental.pallas import tpu_sc as plsc

N = 10000
D = 128
HD = 128
C = 40
E = 320000

NC = 2
NS = 16
NW = NC * NS
CHUNK = 128
NCHUNKS = 80
ACH = 128
ANCH = 80
HANCH = ANCH // 2
NBUF = 2
EDGES_PER_TILE = NCHUNKS * CHUNK
EPAD = EDGES_PER_TILE * NW
NPAD = 10112
ROWS_PER_SUB = NPAD // NS

BT = 512
_GRID = -(-N // BT)

@functools.cache
def _sc_mesh():
    return plsc.VectorSubcoreMesh(core_axis_name="c", subcore_axis_name="s")


def _const_fill(ref, nrows, ncols, value):
    @pl.loop(0, nrows)
    def _(r):
        @pl.loop(0, ncols, step=16)
        def _(c):
            ref[r, pl.ds(c, 16)] = jnp.full((16,), value, jnp.float32)


def _zero_shared_slab(zero_v, acc_sh, base, zrows):
    full, rem = divmod(ROWS_PER_SUB, zrows)
    @pl.loop(0, full)
    def _(i):
        pltpu.sync_copy(zero_v.at[pl.ds(0, zrows)],
                        acc_sh.at[pl.ds(base + i * zrows, zrows)])
    if rem:
        pltpu.sync_copy(zero_v.at[pl.ds(0, rem)],
                        acc_sh.at[pl.ds(base + full * zrows, rem)])


def _drain_shared_slab(acc_sh, out_core, base):
    @pl.loop(0, 4)
    def _(i):
        pltpu.sync_copy(acc_sh.at[pl.ds(base + i * 128, 128)],
                        out_core.at[pl.ds(base + i * 128, 128)])
    pltpu.sync_copy(acc_sh.at[pl.ds(base + 4 * 128, 120)],
                    out_core.at[pl.ds(base + 4 * 128, 120)])


@functools.cache
def _sc_degree_kernel():
    return pl.kernel(
        _sc_degree_body,
        out_type=jax.ShapeDtypeStruct((NC, NPAD, HD), jnp.float32),
        mesh=_sc_mesh(),
        scratch_types=[
            pltpu.VMEM((NCHUNKS, CHUNK), jnp.int32),
            pltpu.VMEM((CHUNK, HD), jnp.float32),
            pltpu.VMEM_SHARED((NPAD, HD), jnp.float32),
        ],
    )


def _sc_degree(dst3):
    return _sc_degree_kernel()(dst3)


def _sc_degree_body(dst_hbm, out_hbm, didx_v, ones_v, acc_sh):
    cid = lax.axis_index("c")
    sid = lax.axis_index("s")
    wid = sid * NC + cid
    base = sid * ROWS_PER_SUB
    _const_fill(ones_v, CHUNK, HD, 0.0)
    _zero_shared_slab(ones_v, acc_sh, base, CHUNK)
    _const_fill(ones_v, CHUNK, HD, 1.0)
    pltpu.sync_copy(dst_hbm.at[wid], didx_v)
    plsc.subcore_barrier()

    @pl.loop(0, NCHUNKS)
    def _(i):
        pltpu.sync_copy(ones_v, acc_sh.at[didx_v.at[i]], add=True)
    plsc.subcore_barrier()

    _drain_shared_slab(acc_sh, out_hbm.at[cid], base)


@functools.cache
def _sc_aggregate_kernel():
    return pl.kernel(
        _sc_aggregate_body,
        out_type=jax.ShapeDtypeStruct((NC, NPAD, HD), jnp.float32),
        mesh=_sc_mesh(),
        scratch_types=[
            pltpu.VMEM((HANCH, ACH), jnp.int32),
            pltpu.VMEM((HANCH, ACH), jnp.int32),
            *[pltpu.VMEM((ACH, HD), jnp.float32) for _ in range(NBUF)],
            pltpu.VMEM_SHARED((NPAD, HD), jnp.float32),
            *[pltpu.SemaphoreType.DMA for _ in range(NBUF)],
        ],
    )


def _sc_aggregate(g, src3, dst3):
    return _sc_aggregate_kernel()(g, src3, dst3)


def _sc_aggregate_body(g_hbm, src_hbm, dst_hbm, out_hbm,
                       src_v, dst_v, r0, r1, acc_sh, g0, g1):
    rows = (r0, r1)
    gsem = (g0, g1)
    cid = lax.axis_index("c")
    sid = lax.axis_index("s")
    wid = sid * NC + cid
    base = sid * ROWS_PER_SUB

    def _wait_gather(buf, sem):
        pltpu.make_async_copy(g_hbm.at[pl.ds(0, ACH)], buf, sem).wait()

    _const_fill(r0, ACH, HD, 0.0)
    _zero_shared_slab(r0, acc_sh, base, ACH)
    plsc.subcore_barrier()

    for h in range(2):
        pltpu.sync_copy(src_hbm.at[wid, pl.ds(h * HANCH, HANCH)], src_v)
        pltpu.sync_copy(dst_hbm.at[wid, pl.ds(h * HANCH, HANCH)], dst_v)
        for b in range(NBUF):
            pltpu.async_copy(g_hbm.at[src_v.at[b]], rows[b], gsem[b])

        @pl.loop(0, HANCH, step=NBUF)
        def _(i):
            for b in range(NBUF):
                _wait_gather(rows[b], gsem[b])
                pltpu.sync_copy(rows[b], acc_sh.at[dst_v.at[i + b]], add=True)
                la = i + b + NBUF
                @pl.when(la < HANCH)
                def _():
                    pltpu.async_copy(g_hbm.at[src_v.at[la]], rows[b], gsem[b])
    plsc.subcore_barrier()

    _drain_shared_slab(acc_sh, out_hbm.at[cid], base)


def _tc_matmul(x, w):
    def body(x_ref, w_ref, o_ref):
        o_ref[...] = jnp.dot(x_ref[...], w_ref[...],
                             preferred_element_type=jnp.float32)
    return pl.pallas_call(
        body,
        grid=(_GRID,),
        in_specs=[pl.BlockSpec((BT, x.shape[1]), lambda i: (i, 0)),
                  pl.BlockSpec(w.shape, lambda i: (0, 0))],
        out_specs=pl.BlockSpec((BT, w.shape[1]), lambda i: (i, 0)),
        out_shape=jax.ShapeDtypeStruct((N, w.shape[1]), jnp.float32),
    )(x, w)


def _tc_scale(h, deg0, deg1):
    def body(h_ref, d0_ref, d1_ref, g_ref, dv_ref):
        deg = d0_ref[...] + d1_ref[...] + 1.0
        dinv = lax.rsqrt(deg)
        g_ref[...] = h_ref[...] * dinv[:, :1]
        dv_ref[...] = dinv
    return pl.pallas_call(
        body,
        grid=(_GRID,),
        in_specs=[pl.BlockSpec((BT, HD), lambda i: (i, 0)),
                  pl.BlockSpec((BT, 16), lambda i: (i, 0)),
                  pl.BlockSpec((BT, 16), lambda i: (i, 0))],
        out_specs=[pl.BlockSpec((BT, HD), lambda i: (i, 0)),
                   pl.BlockSpec((BT, 16), lambda i: (i, 0))],
        out_shape=[jax.ShapeDtypeStruct((N, HD), jnp.float32),
                   jax.ShapeDtypeStruct((N, 16), jnp.float32)],
    )(h, deg0, deg1)


def _tc_combine_matmul(p0, p1, h, dv, b, w):
    def body(p0_ref, p1_ref, h_ref, dv_ref, b_ref, w_ref, hn_ref, gn_ref):
        dinv = dv_ref[...][:, :1]
        pre = (dinv * (p0_ref[...] + p1_ref[...])
               + (dinv * dinv) * h_ref[...] + b_ref[...])
        a = jnp.maximum(pre, 0.0)
        hn = jnp.dot(a, w_ref[...], preferred_element_type=jnp.float32)
        hn_ref[...] = hn
        gn_ref[...] = hn * dinv
    return pl.pallas_call(
        body,
        grid=(_GRID,),
        in_specs=[pl.BlockSpec((BT, HD), lambda i: (i, 0)),
                  pl.BlockSpec((BT, HD), lambda i: (i, 0)),
                  pl.BlockSpec((BT, HD), lambda i: (i, 0)),
                  pl.BlockSpec((BT, 16), lambda i: (i, 0)),
                  pl.BlockSpec((HD,), lambda i: (0,)),
                  pl.BlockSpec((HD, HD), lambda i: (0, 0))],
        out_specs=[pl.BlockSpec((BT, HD), lambda i: (i, 0)),
                   pl.BlockSpec((BT, HD), lambda i: (i, 0))],
        out_shape=[jax.ShapeDtypeStruct((N, HD), jnp.float32),
                   jax.ShapeDtypeStruct((N, HD), jnp.float32)],
    )(p0, p1, h, dv, b, w)


def _tc_head(p0, p1, h, dv, b, wout, bout):
    def body(p0_ref, p1_ref, h_ref, dv_ref, b_ref, w_ref, bo_ref, o_ref):
        dinv = dv_ref[...][:, :1]
        pre = (dinv * (p0_ref[...] + p1_ref[...])
               + (dinv * dinv) * h_ref[...] + b_ref[...])
        a = jnp.maximum(pre, 0.0)
        logits = jnp.dot(a, w_ref[...],
                         preferred_element_type=jnp.float32) + bo_ref[...]
        m = jnp.max(logits, axis=1, keepdims=True)
        ex = jnp.exp(logits - m)
        o_ref[...] = ex / jnp.sum(ex, axis=1, keepdims=True)
    return pl.pallas_call(
        body,
        grid=(_GRID,),
        in_specs=[pl.BlockSpec((BT, HD), lambda i: (i, 0)),
                  pl.BlockSpec((BT, HD), lambda i: (i, 0)),
                  pl.BlockSpec((BT, HD), lambda i: (i, 0)),
                  pl.BlockSpec((BT, 16), lambda i: (i, 0)),
                  pl.BlockSpec((HD,), lambda i: (0,)),
                  pl.BlockSpec((HD, C), lambda i: (0, 0)),
                  pl.BlockSpec((C,), lambda i: (0,))],
        out_specs=pl.BlockSpec((BT, C), lambda i: (i, 0)),
        out_shape=jax.ShapeDtypeStruct((N, C), jnp.float32),
    )(p0, p1, h, dv, b, wout, bout)


def kernel(X, edges_index, W1, b1, W2, b2, W3, b3, Wout, bout):
    src = edges_index[0].astype(jnp.int32)
    dst = edges_index[1].astype(jnp.int32)
    src = jnp.concatenate([src, jnp.zeros((EPAD - E,), jnp.int32)])
    dst = jnp.concatenate([dst, jnp.full((EPAD - E,), N, jnp.int32)])
    src = src.reshape(NW, ANCH, ACH)
    dst = dst.reshape(NW, ANCH, ACH)
    srca, dsta = src, dst

    degp = _sc_degree(dst)
    h1 = _tc_matmul(X, W1)
    g1, dv = _tc_scale(h1, degp[0, :N, :16], degp[1, :N, :16])

    p = _sc_aggregate(g1, srca, dsta)
    h2, g2 = _tc_combine_matmul(p[0, :N], p[1, :N], h1, dv, b1, W2)
    p = _sc_aggregate(g2, srca, dsta)
    h3, g3 = _tc_combine_matmul(p[0, :N], p[1, :N], h2, dv, b2, W3)
    p = _sc_aggregate(g3, srca, dsta)
    return _tc_head(p[0, :N], p[1, :N], h3, dv, b3, Wout, bout)


# speedup vs baseline: 1.0305x; 1.0305x over previous
import functools

import jax
import jax.numpy as jnp
from jax import lax
from jax.experimental import pallas as pl
from jax.experimental.pallas import tpu as pltpu
from jax.experimental.pallas import tpu_sc as plsc

N = 10000
D = 128
HD = 128
C = 40
E = 320000

NC = 2
NS = 16
NW = NC * NS
CHUNK = 128
NCHUNKS = 80
ACH = 64
ANCH = 160
HANCH = ANCH // 4
NBUF = 4
EDGES_PER_TILE = NCHUNKS * CHUNK
EPAD = EDGES_PER_TILE * NW
NPAD = 10112
ROWS_PER_SUB = NPAD // NS

BT = 512
_GRID = -(-N // BT)

@functools.cache
def _sc_mesh():
    return plsc.VectorSubcoreMesh(core_axis_name="c", subcore_axis_name="s")


def _const_fill(ref, nrows, ncols, value):
    @pl.loop(0, nrows)
    def _(r):
        @pl.loop(0, ncols, step=16)
        def _(c):
            ref[r, pl.ds(c, 16)] = jnp.full((16,), value, jnp.float32)


def _zero_shared_slab(zero_v, acc_sh, base, zrows):
    full, rem = divmod(ROWS_PER_SUB, zrows)
    @pl.loop(0, full)
    def _(i):
        pltpu.sync_copy(zero_v.at[pl.ds(0, zrows)],
                        acc_sh.at[pl.ds(base + i * zrows, zrows)])
    if rem:
        pltpu.sync_copy(zero_v.at[pl.ds(0, rem)],
                        acc_sh.at[pl.ds(base + full * zrows, rem)])


def _drain_shared_slab(acc_sh, out_core, base):
    @pl.loop(0, 4)
    def _(i):
        pltpu.sync_copy(acc_sh.at[pl.ds(base + i * 128, 128)],
                        out_core.at[pl.ds(base + i * 128, 128)])
    pltpu.sync_copy(acc_sh.at[pl.ds(base + 4 * 128, 120)],
                    out_core.at[pl.ds(base + 4 * 128, 120)])


@functools.cache
def _sc_degree_kernel():
    return pl.kernel(
        _sc_degree_body,
        out_type=jax.ShapeDtypeStruct((NC, NPAD, HD), jnp.float32),
        mesh=_sc_mesh(),
        scratch_types=[
            pltpu.VMEM((NCHUNKS, CHUNK), jnp.int32),
            pltpu.VMEM((CHUNK, HD), jnp.float32),
            pltpu.VMEM_SHARED((NPAD, HD), jnp.float32),
        ],
    )


def _sc_degree(dst3):
    return _sc_degree_kernel()(dst3)


def _sc_degree_body(dst_hbm, out_hbm, didx_v, ones_v, acc_sh):
    cid = lax.axis_index("c")
    sid = lax.axis_index("s")
    wid = sid * NC + cid
    base = sid * ROWS_PER_SUB
    _const_fill(ones_v, CHUNK, HD, 0.0)
    _zero_shared_slab(ones_v, acc_sh, base, CHUNK)
    _const_fill(ones_v, CHUNK, HD, 1.0)
    pltpu.sync_copy(dst_hbm.at[wid], didx_v)
    plsc.subcore_barrier()

    @pl.loop(0, NCHUNKS)
    def _(i):
        pltpu.sync_copy(ones_v, acc_sh.at[didx_v.at[i]], add=True)
    plsc.subcore_barrier()

    _drain_shared_slab(acc_sh, out_hbm.at[cid], base)


@functools.cache
def _sc_aggregate_kernel():
    return pl.kernel(
        _sc_aggregate_body,
        out_type=jax.ShapeDtypeStruct((NC, NPAD, HD), jnp.float32),
        mesh=_sc_mesh(),
        scratch_types=[
            pltpu.VMEM((HANCH, ACH), jnp.int32),
            pltpu.VMEM((HANCH, ACH), jnp.int32),
            *[pltpu.VMEM((ACH, HD), jnp.float32) for _ in range(NBUF)],
            pltpu.VMEM_SHARED((NPAD, HD), jnp.float32),
            *[pltpu.SemaphoreType.DMA for _ in range(NBUF)],
        ],
    )


def _sc_aggregate(g, src3, dst3):
    return _sc_aggregate_kernel()(g, src3, dst3)


def _sc_aggregate_body(g_hbm, src_hbm, dst_hbm, out_hbm,
                       src_v, dst_v, r0, r1, r2, r3, acc_sh, g0, g1, g2, g3):
    rows = (r0, r1, r2, r3)
    gsem = (g0, g1, g2, g3)
    cid = lax.axis_index("c")
    sid = lax.axis_index("s")
    wid = sid * NC + cid
    base = sid * ROWS_PER_SUB

    def _wait_gather(buf, sem):
        pltpu.make_async_copy(g_hbm.at[pl.ds(0, ACH)], buf, sem).wait()

    _const_fill(r0, ACH, HD, 0.0)
    _zero_shared_slab(r0, acc_sh, base, ACH)
    plsc.subcore_barrier()

    for h in range(ANCH // HANCH):
        pltpu.sync_copy(src_hbm.at[wid, pl.ds(h * HANCH, HANCH)], src_v)
        pltpu.sync_copy(dst_hbm.at[wid, pl.ds(h * HANCH, HANCH)], dst_v)
        for b in range(NBUF):
            pltpu.async_copy(g_hbm.at[src_v.at[b]], rows[b], gsem[b])

        @pl.loop(0, HANCH, step=NBUF)
        def _(i):
            for b in range(NBUF):
                _wait_gather(rows[b], gsem[b])
                pltpu.sync_copy(rows[b], acc_sh.at[dst_v.at[i + b]], add=True)
                la = i + b + NBUF
                @pl.when(la < HANCH)
                def _():
                    pltpu.async_copy(g_hbm.at[src_v.at[la]], rows[b], gsem[b])
    plsc.subcore_barrier()

    _drain_shared_slab(acc_sh, out_hbm.at[cid], base)


def _tc_matmul(x, w):
    def body(x_ref, w_ref, o_ref):
        o_ref[...] = jnp.dot(x_ref[...], w_ref[...],
                             preferred_element_type=jnp.float32)
    return pl.pallas_call(
        body,
        grid=(_GRID,),
        in_specs=[pl.BlockSpec((BT, x.shape[1]), lambda i: (i, 0)),
                  pl.BlockSpec(w.shape, lambda i: (0, 0))],
        out_specs=pl.BlockSpec((BT, w.shape[1]), lambda i: (i, 0)),
        out_shape=jax.ShapeDtypeStruct((N, w.shape[1]), jnp.float32),
    )(x, w)


def _tc_scale(h, deg0, deg1):
    def body(h_ref, d0_ref, d1_ref, g_ref, dv_ref):
        deg = d0_ref[...] + d1_ref[...] + 1.0
        dinv = lax.rsqrt(deg)
        g_ref[...] = h_ref[...] * dinv[:, :1]
        dv_ref[...] = dinv
    return pl.pallas_call(
        body,
        grid=(_GRID,),
        in_specs=[pl.BlockSpec((BT, HD), lambda i: (i, 0)),
                  pl.BlockSpec((BT, 16), lambda i: (i, 0)),
                  pl.BlockSpec((BT, 16), lambda i: (i, 0))],
        out_specs=[pl.BlockSpec((BT, HD), lambda i: (i, 0)),
                   pl.BlockSpec((BT, 16), lambda i: (i, 0))],
        out_shape=[jax.ShapeDtypeStruct((N, HD), jnp.float32),
                   jax.ShapeDtypeStruct((N, 16), jnp.float32)],
    )(h, deg0, deg1)


def _tc_combine_matmul(p0, p1, h, dv, b, w):
    def body(p0_ref, p1_ref, h_ref, dv_ref, b_ref, w_ref, hn_ref, gn_ref):
        dinv = dv_ref[...][:, :1]
        pre = (dinv * (p0_ref[...] + p1_ref[...])
               + (dinv * dinv) * h_ref[...] + b_ref[...])
        a = jnp.maximum(pre, 0.0)
        hn = jnp.dot(a, w_ref[...], preferred_element_type=jnp.float32)
        hn_ref[...] = hn
        gn_ref[...] = hn * dinv
    return pl.pallas_call(
        body,
        grid=(_GRID,),
        in_specs=[pl.BlockSpec((BT, HD), lambda i: (i, 0)),
                  pl.BlockSpec((BT, HD), lambda i: (i, 0)),
                  pl.BlockSpec((BT, HD), lambda i: (i, 0)),
                  pl.BlockSpec((BT, 16), lambda i: (i, 0)),
                  pl.BlockSpec((HD,), lambda i: (0,)),
                  pl.BlockSpec((HD, HD), lambda i: (0, 0))],
        out_specs=[pl.BlockSpec((BT, HD), lambda i: (i, 0)),
                   pl.BlockSpec((BT, HD), lambda i: (i, 0))],
        out_shape=[jax.ShapeDtypeStruct((N, HD), jnp.float32),
                   jax.ShapeDtypeStruct((N, HD), jnp.float32)],
    )(p0, p1, h, dv, b, w)


def _tc_head(p0, p1, h, dv, b, wout, bout):
    def body(p0_ref, p1_ref, h_ref, dv_ref, b_ref, w_ref, bo_ref, o_ref):
        dinv = dv_ref[...][:, :1]
        pre = (dinv * (p0_ref[...] + p1_ref[...])
               + (dinv * dinv) * h_ref[...] + b_ref[...])
        a = jnp.maximum(pre, 0.0)
        logits = jnp.dot(a, w_ref[...],
                         preferred_element_type=jnp.float32) + bo_ref[...]
        m = jnp.max(logits, axis=1, keepdims=True)
        ex = jnp.exp(logits - m)
        o_ref[...] = ex / jnp.sum(ex, axis=1, keepdims=True)
    return pl.pallas_call(
        body,
        grid=(_GRID,),
        in_specs=[pl.BlockSpec((BT, HD), lambda i: (i, 0)),
                  pl.BlockSpec((BT, HD), lambda i: (i, 0)),
                  pl.BlockSpec((BT, HD), lambda i: (i, 0)),
                  pl.BlockSpec((BT, 16), lambda i: (i, 0)),
                  pl.BlockSpec((HD,), lambda i: (0,)),
                  pl.BlockSpec((HD, C), lambda i: (0, 0)),
                  pl.BlockSpec((C,), lambda i: (0,))],
        out_specs=pl.BlockSpec((BT, C), lambda i: (i, 0)),
        out_shape=jax.ShapeDtypeStruct((N, C), jnp.float32),
    )(p0, p1, h, dv, b, wout, bout)


def kernel(X, edges_index, W1, b1, W2, b2, W3, b3, Wout, bout):
    src = edges_index[0].astype(jnp.int32)
    dst = edges_index[1].astype(jnp.int32)
    src = jnp.concatenate([src, jnp.zeros((EPAD - E,), jnp.int32)])
    dst = jnp.concatenate([dst, jnp.full((EPAD - E,), N, jnp.int32)])
    srca = src.reshape(NW, ANCH, ACH)
    dsta = dst.reshape(NW, ANCH, ACH)
    dst = dst.reshape(NW, NCHUNKS, CHUNK)

    degp = _sc_degree(dst)
    h1 = _tc_matmul(X, W1)
    g1, dv = _tc_scale(h1, degp[0, :N, :16], degp[1, :N, :16])

    p = _sc_aggregate(g1, srca, dsta)
    h2, g2 = _tc_combine_matmul(p[0, :N], p[1, :N], h1, dv, b1, W2)
    p = _sc_aggregate(g2, srca, dsta)
    h3, g3 = _tc_combine_matmul(p[0, :N], p[1, :N], h2, dv, b2, W3)
    p = _sc_aggregate(g3, srca, dsta)
    return _tc_head(p[0, :N], p[1, :N], h3, dv, b3, Wout, bout)


# speedup vs baseline: 1.0309x; 1.0004x over previous
import functools

import jax
import jax.numpy as jnp
from jax import lax
from jax.experimental import pallas as pl
from jax.experimental.pallas import tpu as pltpu
from jax.experimental.pallas import tpu_sc as plsc

N = 10000
D = 128
HD = 128
C = 40
E = 320000

NC = 2
NS = 16
NW = NC * NS
CHUNK = 128
NCHUNKS = 80
ACH = 64
ANCH = 160
HANCH = ANCH // 4
NBUF = 4
EDGES_PER_TILE = NCHUNKS * CHUNK
EPAD = EDGES_PER_TILE * NW
NPAD = 10112
ROWS_PER_SUB = NPAD // NS

BT = 512
_GRID = -(-N // BT)

@functools.cache
def _sc_mesh():
    return plsc.VectorSubcoreMesh(core_axis_name="c", subcore_axis_name="s")


def _const_fill(ref, nrows, ncols, value):
    @pl.loop(0, nrows)
    def _(r):
        @pl.loop(0, ncols, step=16)
        def _(c):
            ref[r, pl.ds(c, 16)] = jnp.full((16,), value, jnp.float32)


def _zero_shared_slab(zero_v, acc_sh, base, zrows):
    full, rem = divmod(ROWS_PER_SUB, zrows)
    @pl.loop(0, full)
    def _(i):
        pltpu.sync_copy(zero_v.at[pl.ds(0, zrows)],
                        acc_sh.at[pl.ds(base + i * zrows, zrows)])
    if rem:
        pltpu.sync_copy(zero_v.at[pl.ds(0, rem)],
                        acc_sh.at[pl.ds(base + full * zrows, rem)])


def _drain_shared_slab(acc_sh, out_core, base):
    @pl.loop(0, 4)
    def _(i):
        pltpu.sync_copy(acc_sh.at[pl.ds(base + i * 128, 128)],
                        out_core.at[pl.ds(base + i * 128, 128)])
    pltpu.sync_copy(acc_sh.at[pl.ds(base + 4 * 128, 120)],
                    out_core.at[pl.ds(base + 4 * 128, 120)])


DW = 128

@functools.cache
def _sc_degree_kernel():
    return pl.kernel(
        _sc_degree_body,
        out_type=jax.ShapeDtypeStruct((NC, NPAD, DW), jnp.float32),
        mesh=_sc_mesh(),
        scratch_types=[
            pltpu.VMEM((NCHUNKS, CHUNK), jnp.int32),
            pltpu.VMEM((CHUNK, DW), jnp.float32),
            pltpu.VMEM_SHARED((NPAD, DW), jnp.float32),
        ],
    )


def _sc_degree(dst3):
    return _sc_degree_kernel()(dst3)


def _sc_degree_body(dst_hbm, out_hbm, didx_v, ones_v, acc_sh):
    cid = lax.axis_index("c")
    sid = lax.axis_index("s")
    wid = sid * NC + cid
    base = sid * ROWS_PER_SUB
    _const_fill(ones_v, CHUNK, DW, 0.0)
    _zero_shared_slab(ones_v, acc_sh, base, CHUNK)
    _const_fill(ones_v, CHUNK, DW, 1.0)
    pltpu.sync_copy(dst_hbm.at[wid], didx_v)
    plsc.subcore_barrier()

    @pl.loop(0, NCHUNKS)
    def _(i):
        pltpu.sync_copy(ones_v, acc_sh.at[didx_v.at[i]], add=True)
    plsc.subcore_barrier()

    _drain_shared_slab(acc_sh, out_hbm.at[cid], base)


@functools.cache
def _sc_aggregate_kernel():
    return pl.kernel(
        _sc_aggregate_body,
        out_type=jax.ShapeDtypeStruct((NC, NPAD, HD), jnp.float32),
        mesh=_sc_mesh(),
        scratch_types=[
            pltpu.VMEM((HANCH, ACH), jnp.int32),
            pltpu.VMEM((HANCH, ACH), jnp.int32),
            *[pltpu.VMEM((ACH, HD), jnp.float32) for _ in range(NBUF)],
            pltpu.VMEM_SHARED((NPAD, HD), jnp.float32),
            *[pltpu.SemaphoreType.DMA for _ in range(NBUF)],
        ],
    )


def _sc_aggregate(g, src3, dst3):
    return _sc_aggregate_kernel()(g, src3, dst3)


def _sc_aggregate_body(g_hbm, src_hbm, dst_hbm, out_hbm,
                       src_v, dst_v, r0, r1, r2, r3, acc_sh, g0, g1, g2, g3):
    rows = (r0, r1, r2, r3)
    gsem = (g0, g1, g2, g3)
    cid = lax.axis_index("c")
    sid = lax.axis_index("s")
    wid = sid * NC + cid
    base = sid * ROWS_PER_SUB

    def _wait_gather(buf, sem):
        pltpu.make_async_copy(g_hbm.at[pl.ds(0, ACH)], buf, sem).wait()

    _const_fill(r0, ACH, HD, 0.0)
    _zero_shared_slab(r0, acc_sh, base, ACH)
    plsc.subcore_barrier()

    for h in range(ANCH // HANCH):
        pltpu.sync_copy(src_hbm.at[wid, pl.ds(h * HANCH, HANCH)], src_v)
        pltpu.sync_copy(dst_hbm.at[wid, pl.ds(h * HANCH, HANCH)], dst_v)
        for b in range(NBUF):
            pltpu.async_copy(g_hbm.at[src_v.at[b]], rows[b], gsem[b])

        @pl.loop(0, HANCH, step=NBUF)
        def _(i):
            for b in range(NBUF):
                _wait_gather(rows[b], gsem[b])
                pltpu.sync_copy(rows[b], acc_sh.at[dst_v.at[i + b]], add=True)
                la = i + b + NBUF
                @pl.when(la < HANCH)
                def _():
                    pltpu.async_copy(g_hbm.at[src_v.at[la]], rows[b], gsem[b])
    plsc.subcore_barrier()

    _drain_shared_slab(acc_sh, out_hbm.at[cid], base)


def _tc_matmul(x, w):
    def body(x_ref, w_ref, o_ref):
        o_ref[...] = jnp.dot(x_ref[...], w_ref[...],
                             preferred_element_type=jnp.float32)
    return pl.pallas_call(
        body,
        grid=(_GRID,),
        in_specs=[pl.BlockSpec((BT, x.shape[1]), lambda i: (i, 0)),
                  pl.BlockSpec(w.shape, lambda i: (0, 0))],
        out_specs=pl.BlockSpec((BT, w.shape[1]), lambda i: (i, 0)),
        out_shape=jax.ShapeDtypeStruct((N, w.shape[1]), jnp.float32),
    )(x, w)


def _tc_scale(h, deg0, deg1):
    def body(h_ref, d0_ref, d1_ref, g_ref, dv_ref):
        deg = d0_ref[...] + d1_ref[...] + 1.0
        dinv = lax.rsqrt(deg)
        g_ref[...] = h_ref[...] * dinv[:, :1]
        dv_ref[...] = dinv
    return pl.pallas_call(
        body,
        grid=(_GRID,),
        in_specs=[pl.BlockSpec((BT, HD), lambda i: (i, 0)),
                  pl.BlockSpec((BT, 16), lambda i: (i, 0)),
                  pl.BlockSpec((BT, 16), lambda i: (i, 0))],
        out_specs=[pl.BlockSpec((BT, HD), lambda i: (i, 0)),
                   pl.BlockSpec((BT, 16), lambda i: (i, 0))],
        out_shape=[jax.ShapeDtypeStruct((N, HD), jnp.float32),
                   jax.ShapeDtypeStruct((N, 16), jnp.float32)],
    )(h, deg0, deg1)


def _tc_combine_matmul(p0, p1, h, dv, b, w):
    def body(p0_ref, p1_ref, h_ref, dv_ref, b_ref, w_ref, hn_ref, gn_ref):
        dinv = dv_ref[...][:, :1]
        pre = (dinv * (p0_ref[...] + p1_ref[...])
               + (dinv * dinv) * h_ref[...] + b_ref[...])
        a = jnp.maximum(pre, 0.0)
        hn = jnp.dot(a, w_ref[...], preferred_element_type=jnp.float32)
        hn_ref[...] = hn
        gn_ref[...] = hn * dinv
    return pl.pallas_call(
        body,
        grid=(_GRID,),
        in_specs=[pl.BlockSpec((BT, HD), lambda i: (i, 0)),
                  pl.BlockSpec((BT, HD), lambda i: (i, 0)),
                  pl.BlockSpec((BT, HD), lambda i: (i, 0)),
                  pl.BlockSpec((BT, 16), lambda i: (i, 0)),
                  pl.BlockSpec((HD,), lambda i: (0,)),
                  pl.BlockSpec((HD, HD), lambda i: (0, 0))],
        out_specs=[pl.BlockSpec((BT, HD), lambda i: (i, 0)),
                   pl.BlockSpec((BT, HD), lambda i: (i, 0))],
        out_shape=[jax.ShapeDtypeStruct((N, HD), jnp.float32),
                   jax.ShapeDtypeStruct((N, HD), jnp.float32)],
    )(p0, p1, h, dv, b, w)


def _tc_head(p0, p1, h, dv, b, wout, bout):
    def body(p0_ref, p1_ref, h_ref, dv_ref, b_ref, w_ref, bo_ref, o_ref):
        dinv = dv_ref[...][:, :1]
        pre = (dinv * (p0_ref[...] + p1_ref[...])
               + (dinv * dinv) * h_ref[...] + b_ref[...])
        a = jnp.maximum(pre, 0.0)
        logits = jnp.dot(a, w_ref[...],
                         preferred_element_type=jnp.float32) + bo_ref[...]
        m = jnp.max(logits, axis=1, keepdims=True)
        ex = jnp.exp(logits - m)
        o_ref[...] = ex / jnp.sum(ex, axis=1, keepdims=True)
    return pl.pallas_call(
        body,
        grid=(_GRID,),
        in_specs=[pl.BlockSpec((BT, HD), lambda i: (i, 0)),
                  pl.BlockSpec((BT, HD), lambda i: (i, 0)),
                  pl.BlockSpec((BT, HD), lambda i: (i, 0)),
                  pl.BlockSpec((BT, 16), lambda i: (i, 0)),
                  pl.BlockSpec((HD,), lambda i: (0,)),
                  pl.BlockSpec((HD, C), lambda i: (0, 0)),
                  pl.BlockSpec((C,), lambda i: (0,))],
        out_specs=pl.BlockSpec((BT, C), lambda i: (i, 0)),
        out_shape=jax.ShapeDtypeStruct((N, C), jnp.float32),
    )(p0, p1, h, dv, b, wout, bout)


def kernel(X, edges_index, W1, b1, W2, b2, W3, b3, Wout, bout):
    src = edges_index[0].astype(jnp.int32)
    dst = edges_index[1].astype(jnp.int32)
    src = jnp.concatenate([src, jnp.zeros((EPAD - E,), jnp.int32)])
    dst = jnp.concatenate([dst, jnp.full((EPAD - E,), N, jnp.int32)])
    srca = src.reshape(NW, ANCH, ACH)
    dsta = dst.reshape(NW, ANCH, ACH)
    dst = dst.reshape(NW, NCHUNKS, CHUNK)

    degp = _sc_degree(dst)
    h1 = _tc_matmul(X, W1)
    g1, dv = _tc_scale(h1, degp[0, :N, :16], degp[1, :N, :16])

    p = _sc_aggregate(g1, srca, dsta)
    h2, g2 = _tc_combine_matmul(p[0, :N], p[1, :N], h1, dv, b1, W2)
    p = _sc_aggregate(g2, srca, dsta)
    h3, g3 = _tc_combine_matmul(p[0, :N], p[1, :N], h2, dv, b2, W3)
    p = _sc_aggregate(g3, srca, dsta)
    return _tc_head(p[0, :N], p[1, :N], h3, dv, b3, Wout, bout)
